# baseline (speedup 1.0000x reference)
import functools

import jax
import jax.numpy as jnp
from jax import lax
from jax.experimental import pallas as pl
from jax.experimental.pallas import tpu as pltpu
from jax.experimental.pallas import tpu_sc as plsc

EMB = 64
CHUNK = 128
SLAB = 2
SROWS = CHUNK * SLAB


def _gather_body(n_slabs, nc, table_hbm, idx_hbm, out_hbm,
                 idx_v, rows_v, tbuf, sem_g, sem_o, sem_i):
    wid = lax.axis_index("s") * nc + lax.axis_index("c")
    s0 = wid * n_slabs
    iota16 = lax.iota(jnp.int32, 16)

    def stage_idx(t, p):
        pltpu.async_copy(idx_hbm.at[pl.ds((s0 + t) * SLAB, SLAB)],
                         idx_v.at[p], sem_i.at[p])

    def wait_idx(t, p):
        pltpu.make_async_copy(idx_hbm.at[pl.ds((s0 + t) * SLAB, SLAB)],
                              idx_v.at[p], sem_i.at[p]).wait()

    def fire_gathers(p):
        for k in range(SLAB):
            pltpu.async_copy(table_hbm.at[idx_v.at[p, k]],
                             rows_v.at[p, pl.ds(k * CHUNK, CHUNK)],
                             sem_g.at[p])

    def drain_gathers(p):
        pltpu.make_async_copy(table_hbm.at[pl.ds(0, SROWS)], rows_v.at[p],
                              sem_g.at[p]).wait()

    def out_slice(t):
        g = (s0 + t) * SLAB
        return out_hbm.at[g // 128, :, pl.ds(g % 128, SLAB)]

    def start_write(t, p):
        pltpu.async_copy(tbuf.at[p], out_slice(t), sem_o.at[p])

    def drain_write(t, p):
        pltpu.make_async_copy(tbuf.at[p], out_slice(t), sem_o.at[p]).wait()

    def transpose(p):
        rows2d = rows_v.at[p]

        def tbody(cc, carry):
            for c in (2 * cc, 2 * cc + 1):
                col = jnp.full((16,), c, jnp.int32)
                dst_c = tbuf.at[p, c // 8]
                vals = [
                    plsc.load_gather(
                        rows2d, [iota16 + (at * 128 + alb * 16), col])
                    for at in range(SLAB) for alb in range(8)
                ]
                for i, v in enumerate(vals):
                    at, alb = divmod(i, 8)
                    dst_c[at, c % 8, pl.ds(alb * 16, 16)] = v
            return carry

        lax.fori_loop(0, EMB // 2, tbody, 0)

    stage_idx(0, 0)
    stage_idx(1, 1)
    wait_idx(0, 0)
    fire_gathers(0)

    def body(t, carry):
        p = t % 2
        q = 1 - p
        drain_gathers(p)

        @pl.when(t + 1 < n_slabs)
        def _next_gathers():
            wait_idx(t + 1, q)
            fire_gathers(q)

        @pl.when(t >= 2)
        def _free_tbuf():
            drain_write(t - 2, p)

        transpose(p)
        start_write(t, p)

        @pl.when(t + 2 < n_slabs)
        def _next_idx():
            stage_idx(t + 2, p)
        return carry

    lax.fori_loop(0, n_slabs, body, 0)
    drain_write(n_slabs - 2, n_slabs % 2)
    drain_write(n_slabs - 1, (n_slabs - 1) % 2)


@jax.jit
def kernel(inputs, weight):
    n_a, n_b = inputs.shape
    n_idx = n_a * n_b
    idx2d = inputs.T.astype(jnp.int32).reshape(n_idx // CHUNK, CHUNK)
    table = weight

    info = plsc.get_sparse_core_info()
    nc, ns = info.num_cores, info.num_subcores
    nw = nc * ns
    n_slabs = n_idx // SROWS // nw

    mesh = plsc.VectorSubcoreMesh(core_axis_name="c", subcore_axis_name="s")
    out5 = pl.kernel(
        functools.partial(_gather_body, n_slabs, nc),
        out_type=jax.ShapeDtypeStruct(
            (n_b, EMB // 8, n_a // CHUNK, 8, CHUNK), jnp.float32),
        mesh=mesh,
        scratch_types=[
            pltpu.VMEM((2, SLAB, CHUNK), jnp.int32),
            pltpu.VMEM((2, SROWS, EMB), jnp.float32),
            pltpu.VMEM((2, EMB // 8, SLAB, 8, CHUNK), jnp.float32),
            pltpu.SemaphoreType.DMA((2,)),
            pltpu.SemaphoreType.DMA((2,)),
            pltpu.SemaphoreType.DMA((2,)),
        ],
        compiler_params=pltpu.CompilerParams(use_tc_tiling_on_sc=False,
                                             needs_layout_passes=False),
    )(table, idx2d)
    out = jnp.transpose(out5, (2, 4, 0, 1, 3)).reshape(n_a, n_b, EMB)
    return out


# speedup vs baseline: 1.0008x; 1.0008x over previous
import functools

import jax
import jax.numpy as jnp
from jax import lax
from jax.experimental import pallas as pl
from jax.experimental.pallas import tpu as pltpu
from jax.experimental.pallas import tpu_sc as plsc

EMB = 64
CHUNK = 128
SLAB = 2
SROWS = CHUNK * SLAB


def _gather_body(n_slabs, nc, table_hbm, idx_hbm, out_hbm,
                 idx_v, rows_v, tbuf, sem_g, sem_o, sem_i):
    wid = lax.axis_index("s") * nc + lax.axis_index("c")
    s0 = wid * n_slabs
    iota16 = lax.iota(jnp.int32, 16)

    def stage_idx(t, p):
        pltpu.async_copy(idx_hbm.at[pl.ds((s0 + t) * SLAB, SLAB)],
                         idx_v.at[p], sem_i.at[p])

    def wait_idx(t, p):
        pltpu.make_async_copy(idx_hbm.at[pl.ds((s0 + t) * SLAB, SLAB)],
                              idx_v.at[p], sem_i.at[p]).wait()

    def fire_gathers(p):
        for k in range(SLAB):
            pltpu.async_copy(table_hbm.at[idx_v.at[p, k]],
                             rows_v.at[p, pl.ds(k * CHUNK, CHUNK)],
                             sem_g.at[p])

    def drain_gathers(p):
        pltpu.make_async_copy(table_hbm.at[pl.ds(0, SROWS)], rows_v.at[p],
                              sem_g.at[p]).wait()

    def out_slice(t):
        g = (s0 + t) * SLAB
        return out_hbm.at[g // 128, :, pl.ds(g % 128, SLAB)]

    def start_write(t, p):
        pltpu.async_copy(tbuf.at[p], out_slice(t), sem_o.at[p])

    def drain_write(t, p):
        pltpu.make_async_copy(tbuf.at[p], out_slice(t), sem_o.at[p]).wait()

    def transpose(p, pt):
        rows2d = rows_v.at[p]

        def tbody(cc, carry):
            for c in (2 * cc, 2 * cc + 1):
                col = jnp.full((16,), c, jnp.int32)
                dst_c = tbuf.at[pt, c // 8]
                vals = [
                    plsc.load_gather(
                        rows2d, [iota16 + (at * 128 + alb * 16), col])
                    for at in range(SLAB) for alb in range(8)
                ]
                for i, v in enumerate(vals):
                    at, alb = divmod(i, 8)
                    dst_c[at, c % 8, pl.ds(alb * 16, 16)] = v
            return carry

        lax.fori_loop(0, EMB // 2, tbody, 0)

    stage_idx(0, 0)
    stage_idx(1, 1)
    stage_idx(2, 2)
    wait_idx(0, 0)
    fire_gathers(0)
    wait_idx(1, 1)
    fire_gathers(1)

    def body(t, carry):
        pr = t % 3
        pt = t % 2
        drain_gathers(pr)

        @pl.when(t + 2 < n_slabs)
        def _next_gathers():
            pr2 = (t + 2) % 3
            wait_idx(t + 2, pr2)
            fire_gathers(pr2)

        @pl.when(t >= 2)
        def _free_tbuf():
            drain_write(t - 2, pt)

        transpose(pr, pt)
        start_write(t, pt)

        @pl.when(t + 3 < n_slabs)
        def _next_idx():
            stage_idx(t + 3, pr)
        return carry

    lax.fori_loop(0, n_slabs, body, 0)
    drain_write(n_slabs - 2, n_slabs % 2)
    drain_write(n_slabs - 1, (n_slabs - 1) % 2)


@jax.jit
def kernel(inputs, weight):
    n_a, n_b = inputs.shape
    n_idx = n_a * n_b
    idx2d = inputs.T.astype(jnp.int32).reshape(n_idx // CHUNK, CHUNK)
    table = weight

    info = plsc.get_sparse_core_info()
    nc, ns = info.num_cores, info.num_subcores
    nw = nc * ns
    n_slabs = n_idx // SROWS // nw

    mesh = plsc.VectorSubcoreMesh(core_axis_name="c", subcore_axis_name="s")
    out5 = pl.kernel(
        functools.partial(_gather_body, n_slabs, nc),
        out_type=jax.ShapeDtypeStruct(
            (n_b, EMB // 8, n_a // CHUNK, 8, CHUNK), jnp.float32),
        mesh=mesh,
        scratch_types=[
            pltpu.VMEM((3, SLAB, CHUNK), jnp.int32),
            pltpu.VMEM((3, SROWS, EMB), jnp.float32),
            pltpu.VMEM((2, EMB // 8, SLAB, 8, CHUNK), jnp.float32),
            pltpu.SemaphoreType.DMA((3,)),
            pltpu.SemaphoreType.DMA((2,)),
            pltpu.SemaphoreType.DMA((3,)),
        ],
        compiler_params=pltpu.CompilerParams(use_tc_tiling_on_sc=False,
                                             needs_layout_passes=False),
    )(table, idx2d)
    out = jnp.transpose(out5, (2, 4, 0, 1, 3)).reshape(n_a, n_b, EMB)
    return out
